# Initial kernel scaffold; baseline (speedup 1.0000x reference)
import jax
import jax.numpy as jnp
from jax.experimental import pallas as pl


def kernel(output1, output2, output3, output4, labels):
    raise NotImplementedError("write your pallas kernel here")


# speedup vs baseline: 58.3035x; 58.3035x over previous
import functools

import jax
import jax.numpy as jnp
from jax.experimental import pallas as pl

_N = 786432
_LANES = 128
_ROWS = _N // _LANES
_BLK = 512
_GRID = _ROWS // _BLK
_NCLS = 4
_RPC = 9


def _bce_terms(x):
    prob = jax.nn.sigmoid(x)
    logp = jnp.maximum(jnp.log(prob), -100.0)
    log1mp = jnp.maximum(jnp.log(1.0 - prob), -100.0)
    return prob, logp, log1mp


def _smooth_l1(d):
    a = jnp.abs(d)
    return jnp.where(a < 1.0, 0.5 * d * d, a - 0.5)


def _acc_body(o1, o2, o3, o4, lab, out_ref):
    i = pl.program_id(0)

    @pl.when(i == 0)
    def _():
        out_ref[...] = jnp.zeros_like(out_ref)

    l0 = lab[0]
    negf = (l0 == -1.0).astype(jnp.float32)

    def rsum(v):
        return jnp.sum(v, axis=0, keepdims=True)

    rows = []
    for c, oc in enumerate((o1, o2, o3, o4), start=1):
        x = oc[0]
        prob, logp, log1mp = _bce_terms(x)
        posf = (l0 == float(c)).astype(jnp.float32)
        rows.append(rsum(posf))
        rows.append(rsum(posf * (-logp)))
        rows.append(rsum(posf * (prob >= 0.5)))
        for j in range(1, 5):
            e = _smooth_l1(oc[j] - lab[j])
            rows.append(rsum(posf * e))
        rows.append(rsum(negf * (-log1mp)))
        rows.append(rsum(negf * (prob < 0.5)))
    rows.append(rsum(negf))
    zero = jnp.zeros((1, _LANES), jnp.float32)
    rows += [zero, zero, zero]
    acc = jnp.concatenate(rows, axis=0)
    out_ref[...] += acc


def _monotone_key(x):
    b = jax.lax.bitcast_convert_type(x, jnp.int32)
    flipped = jnp.bitwise_xor(jnp.bitwise_not(b), jnp.int32(-2**31))
    return jnp.where(b >= 0, b, flipped)


def _select_body(keys_ref, kvals_ref, out_ref):
    out_ref[...] = jnp.zeros_like(out_ref)
    for c in range(_NCLS):
        x = keys_ref[c]
        k = kvals_ref[0, c]
        s = _monotone_key(x)
        shigh = jax.lax.shift_right_arithmetic(s, 16)
        slow = jnp.bitwise_and(s, jnp.int32(0xFFFF))

        def cnt_high(th):
            return jnp.sum((shigh >= th).astype(jnp.float32))

        def hbody(j, hb):
            bit = jax.lax.shift_left(jnp.int32(1), jnp.int32(15) - j)
            trial = jnp.bitwise_or(hb, bit)
            ok = cnt_high(trial - 32768) >= k
            return jnp.where(ok, trial, hb)

        hbits = jax.lax.fori_loop(0, 16, hbody, jnp.int32(0))
        h = hbits - 32768
        cnt_gt_high = jnp.sum((shigh > h).astype(jnp.float32))
        need = k - cnt_gt_high
        in_grp = shigh == h

        def lbody(j, lb):
            bit = jax.lax.shift_left(jnp.int32(1), jnp.int32(15) - j)
            trial = jnp.bitwise_or(lb, bit)
            cnt = jnp.sum((in_grp & (slow >= trial)).astype(jnp.float32))
            return jnp.where(cnt >= need, trial, lb)

        lbits = jax.lax.fori_loop(0, 16, lbody, jnp.int32(0))
        t = jnp.bitwise_or(jax.lax.shift_left(h, jnp.int32(16)), lbits)

        gt = s > t
        _, _, log1mp = _bce_terms(x)
        e = -log1mp
        sum_gt = jnp.sum(jnp.where(gt, e, 0.0))
        cnt_gt = jnp.sum(gt.astype(jnp.float32))
        corr_gt = jnp.sum((gt & (x < 0.0)).astype(jnp.float32))
        xt = jnp.max(jnp.where(s == t, x, -jnp.inf))
        _, _, log1mp_t = _bce_terms(xt)
        rem = k - cnt_gt
        lane = jax.lax.broadcasted_iota(jnp.int32, (1, _LANES), 1)
        vec = jnp.where(lane == 0, sum_gt + rem * (-log1mp_t), 0.0)
        vec = vec + jnp.where(
            lane == 1, corr_gt + rem * (xt < 0.0).astype(jnp.float32), 0.0
        )
        out_ref[c : c + 1, :] = vec


def _run_select(keys, kvals):
    return pl.pallas_call(
        _select_body,
        grid=(1,),
        in_specs=[
            pl.BlockSpec((_NCLS, _ROWS, _LANES), lambda i: (0, 0, 0)),
            pl.BlockSpec((1, _NCLS), lambda i: (0, 0)),
        ],
        out_specs=pl.BlockSpec((_NCLS, _LANES), lambda i: (0, 0)),
        out_shape=jax.ShapeDtypeStruct((_NCLS, _LANES), jnp.float32),
    )(keys, kvals)


@jax.jit
def kernel(output1, output2, output3, output4, labels):
    outs = [o.reshape(-1, 5) for o in (output1, output2, output3, output4)]
    labels = labels.reshape(-1, 5)
    ts = [o.T.reshape(5, _ROWS, _LANES) for o in outs]
    lt = labels.T.reshape(5, _ROWS, _LANES)

    acc = pl.pallas_call(
        _acc_body,
        grid=(_GRID,),
        in_specs=[pl.BlockSpec((5, _BLK, _LANES), lambda i: (0, i, 0))] * 5,
        out_specs=pl.BlockSpec((40, _LANES), lambda i: (0, 0)),
        out_shape=jax.ShapeDtypeStruct((40, _LANES), jnp.float32),
    )(*ts, lt)

    sums = jnp.sum(acc, axis=1)
    neg_cnt = sums[_NCLS * _RPC].astype(jnp.int32)
    ncf = neg_cnt.astype(jnp.float32)

    pos_cnts, ks = [], []
    for c in range(_NCLS):
        p = sums[c * _RPC].astype(jnp.int32)
        pos_cnts.append(p)
        ks.append(jnp.minimum(2 * jnp.maximum(p, 1), neg_cnt))

    need_select = jnp.any(jnp.stack(ks) < neg_cnt)

    def common(_):
        res = []
        for c in range(_NCLS):
            res.append(jnp.stack([sums[c * _RPC + 7], sums[c * _RPC + 8]]))
        return jnp.stack(res)

    def rare(_):
        kvals = jnp.stack([k.astype(jnp.float32) for k in ks]).reshape(1, 4)
        keys = jnp.stack(
            [jnp.where(lt[0] == -1.0, t[0], -jnp.inf) for t in ts]
        )
        sel = _run_select(keys, kvals)
        return sel[:, :2]

    negres = jax.lax.cond(need_select, rare, common, operand=None)

    loss = classify = 0.0
    regs = [0.0, 0.0, 0.0, 0.0]
    pos_correct = jnp.int32(0)
    pos_total = jnp.int32(0)
    neg_correct = jnp.int32(0)
    neg_total = jnp.int32(0)
    for c in range(_NCLS):
        base = c * _RPC
        batch = jnp.maximum(pos_cnts[c], 1)
        bf = batch.astype(jnp.float32)
        kf = ks[c].astype(jnp.float32)
        bce_pos = sums[base + 1] / bf
        bce_neg = negres[c, 0] / kf
        cls_c = 0.5 * bce_pos + 0.5 * bce_neg
        loss_c = cls_c
        for j in range(4):
            r = sums[base + 3 + j] / bf
            regs[j] = regs[j] + r
            loss_c = loss_c + r
        loss = loss + loss_c
        classify = classify + cls_c
        pos_correct = pos_correct + sums[base + 2].astype(jnp.int32)
        pos_total = pos_total + pos_cnts[c]
        neg_correct = neg_correct + negres[c, 1].astype(jnp.int32)
        neg_total = neg_total + ks[c]

    return (
        (loss / 4.0).astype(jnp.float32),
        (classify / 4.0).astype(jnp.float32),
        (regs[0] / 4.0).astype(jnp.float32),
        (regs[1] / 4.0).astype(jnp.float32),
        (regs[2] / 4.0).astype(jnp.float32),
        (regs[3] / 4.0).astype(jnp.float32),
        pos_correct,
        pos_total,
        neg_correct,
        neg_total,
    )
